# Initial kernel scaffold; baseline (speedup 1.0000x reference)
import jax
import jax.numpy as jnp
from jax.experimental import pallas as pl


def kernel(apig, apig_feat, fcg, fcg_feat, W_a1, b_a1, W_a2, b_a2, W_f1, b_f1, W_f2, b_f2, W_l1, b_l1, W_l2, b_l2, W_attn, b_attn, W_c, b_c):
    raise NotImplementedError("write your pallas kernel here")


# speedup vs baseline: 14.6221x; 14.6221x over previous
import functools

import jax
import jax.numpy as jnp
from jax import lax
from jax.experimental import pallas as pl
from jax.experimental.pallas import tpu as pltpu
from jax.experimental.pallas import tpu_sc as plsc

_N = 10000
_E = 320000
_D = 128
_HD = _D // 2
_U = 64
_CH = 80
_CPT = _E // 16 // _CH
_NPAD = 10240
_RPT = _NPAD // 16
_NB = 10
_BR = _N // _NB
_NBUF = 5

_SDS = jax.ShapeDtypeStruct
_mesh = plsc.VectorSubcoreMesh(core_axis_name="c", subcore_axis_name="s")


_DPT = _NPAD // 16


@functools.partial(
    pl.kernel,
    mesh=_mesh,
    out_type=[_SDS((_NPAD,), jnp.float32)] * 4,
    scratch_types=[
        pltpu.VMEM((_CPT, _CH), jnp.int32),
        pltpu.VMEM((_CPT, _CH), jnp.int32),
        pltpu.VMEM((_NPAD,), jnp.float32),
        pltpu.VMEM((_NPAD,), jnp.float32),
        pltpu.VMEM((16, _DPT), jnp.float32),
        pltpu.VMEM((_DPT,), jnp.float32),
        pltpu.VMEM_SHARED((16, _NPAD), jnp.float32),
        pltpu.VMEM_SHARED((16, _NPAD), jnp.float32),
    ],
    compiler_params=pltpu.CompilerParams(needs_layout_passes=False),
)
def _deg_call(src_a, dst_a, src_f, dst_f,
              out_as, out_ad, out_fs, out_fd,
              srcv, dstv, degs_v, degd_v, red_v, outb_v, sh_s, sh_d):
    c = lax.axis_index("c")
    s = lax.axis_index("s")

    def run(src3d, dst3d, out_s, out_d):
        pltpu.sync_copy(src3d.at[s], srcv)
        pltpu.sync_copy(dst3d.at[s], dstv)
        z16 = jnp.zeros((16,), jnp.float32)

        def zbody(i, carry):
            degs_v[pl.ds(i * 16, 16)] = z16
            degd_v[pl.ds(i * 16, 16)] = z16
            return carry

        lax.fori_loop(0, _NPAD // 16, zbody, 0)
        ones = jnp.ones((16,), jnp.float32)

        def body(r, carry):
            for cc in range(_CH // 16):
                si = srcv[r, pl.ds(cc * 16, 16)]
                di = dstv[r, pl.ds(cc * 16, 16)]
                plsc.addupdate_scatter(degs_v, [si], ones)
                plsc.addupdate_scatter(degd_v, [di], ones)
            return carry

        lax.fori_loop(0, _CPT, body, 0)
        pltpu.sync_copy(degs_v, sh_s.at[s])
        pltpu.sync_copy(degd_v, sh_d.at[s])
        plsc.subcore_barrier()

        def reduce_out(sh, out):
            pltpu.sync_copy(sh.at[:, pl.ds(s * _DPT, _DPT)], red_v)
            for chk in range(_DPT // 16):
                v = red_v[0, pl.ds(chk * 16, 16)]
                for t in range(1, 16):
                    v = v + red_v[t, pl.ds(chk * 16, 16)]
                outb_v[pl.ds(chk * 16, 16)] = v
            pltpu.sync_copy(outb_v, out.at[pl.ds(s * _DPT, _DPT)])

        reduce_out(sh_s, out_s)
        reduce_out(sh_d, out_d)

    @pl.when(c == 0)
    def _():
        run(src_a, dst_a, out_as, out_ad)

    @pl.when(c == 1)
    def _():
        run(src_f, dst_f, out_fs, out_fd)


@functools.partial(
    pl.kernel,
    mesh=_mesh,
    out_type=[_SDS((_NPAD, _HD), jnp.float32)] * 4,
    scratch_types=[
        pltpu.VMEM((_CPT, _CH), jnp.int32),
        pltpu.VMEM((_CPT, _CH), jnp.int32),
        pltpu.VMEM((_CH, _HD), jnp.float32),
    ] + [pltpu.VMEM((_CH, _HD), jnp.float32) for _ in range(_NBUF)]
      + [pltpu.SemaphoreType.DMA for _ in range(2 * _NBUF)]
      + [pltpu.VMEM_SHARED((_NPAD, _HD), jnp.float32)],
    compiler_params=pltpu.CompilerParams(use_tc_tiling_on_sc=False),
)
def _agg_call(xa1, xa2, xf1, xf2, src_a, dst_a, src_f, dst_f,
              out_a1, out_a2, out_f1, out_f2, *scr):
    srcv, dstv, zbuf = scr[0], scr[1], scr[2]
    rows = list(scr[3:3 + _NBUF])
    gsem = list(scr[3 + _NBUF:3 + 2 * _NBUF])
    ssem = list(scr[3 + 2 * _NBUF:3 + 3 * _NBUF])
    acc = scr[3 + 3 * _NBUF]
    c = lax.axis_index("c")
    s = lax.axis_index("s")

    def zero_acc():
        for k in range(_RPT // _CH):
            pltpu.sync_copy(zbuf, acc.at[pl.ds(s * _RPT + k * _CH, _CH)])

    def pipeline(xws):
        pltpu.async_copy(xws.at[srcv.at[0]], rows[0], gsem[0])
        pltpu.async_copy(xws.at[srcv.at[1]], rows[1], gsem[1])

        def body(i, carry):
            for b in range(_NBUF):
                j = _NBUF * i + b
                b2 = (b + 2) % _NBUF

                @pl.when((j >= 3) & (j + 2 < _CPT))
                def _():
                    pltpu.make_async_copy(
                        rows[b2], acc.at[dstv.at[j]], ssem[b2]).wait()

                @pl.when(j + 2 < _CPT)
                def _():
                    pltpu.async_copy(
                        xws.at[srcv.at[j + 2]], rows[b2], gsem[b2])

                pltpu.make_async_copy(
                    xws.at[srcv.at[j]], rows[b], gsem[b]).wait()
                pltpu.async_copy(
                    rows[b], acc.at[dstv.at[j]], ssem[b], add=True)
            return carry

        lax.fori_loop(0, _CPT // _NBUF, body, 0)
        for b in range(_NBUF):
            pltpu.make_async_copy(
                rows[b], acc.at[dstv.at[0]], ssem[b]).wait()

    def run(xws1, xws2, src3d, dst3d, out1, out2):
        pltpu.sync_copy(src3d.at[s], srcv)
        pltpu.sync_copy(dst3d.at[s], dstv)
        z16 = jnp.zeros((16,), jnp.float32)

        def zbody(r, carry):
            for cc in range(_HD // 16):
                zbuf[r, pl.ds(cc * 16, 16)] = z16
            return carry

        lax.fori_loop(0, _CH, zbody, 0)
        zero_acc()
        plsc.subcore_barrier()
        pipeline(xws1)
        plsc.subcore_barrier()
        pltpu.sync_copy(acc.at[pl.ds(s * _RPT, _RPT)],
                        out1.at[pl.ds(s * _RPT, _RPT)])
        zero_acc()
        plsc.subcore_barrier()
        pipeline(xws2)
        plsc.subcore_barrier()
        pltpu.sync_copy(acc.at[pl.ds(s * _RPT, _RPT)],
                        out2.at[pl.ds(s * _RPT, _RPT)])

    @pl.when(c == 0)
    def _():
        run(xa1, xa2, src_a, dst_a, out_a1, out_a2)

    @pl.when(c == 1)
    def _():
        run(xf1, xf2, src_f, dst_f, out_f1, out_f2)


def _t1_body(xa_ref, wa_ref, dega_ref, xf_ref, wf_ref, degf_ref,
             oa_ref, of_ref):
    sa = lax.rsqrt(jnp.maximum(dega_ref[0, 0, :], 1.0))
    xwa = jnp.dot(xa_ref[...], wa_ref[...], preferred_element_type=jnp.float32)
    oa_ref[...] = xwa * sa[:, None]
    sf = lax.rsqrt(jnp.maximum(degf_ref[0, 0, :], 1.0))
    xwf = jnp.dot(xf_ref[...], wf_ref[...], preferred_element_type=jnp.float32)
    of_ref[...] = xwf * sf[:, None]


def _t1(xa, wa, dega3, xf, wf, degf3):
    return pl.pallas_call(
        _t1_body,
        grid=(_NB,),
        in_specs=[
            pl.BlockSpec((_BR, _D), lambda i: (i, 0)),
            pl.BlockSpec((_D, _D), lambda i: (0, 0)),
            pl.BlockSpec((1, 1, _BR), lambda i: (i, 0, 0)),
            pl.BlockSpec((_BR, _D), lambda i: (i, 0)),
            pl.BlockSpec((_D, _D), lambda i: (0, 0)),
            pl.BlockSpec((1, 1, _BR), lambda i: (i, 0, 0)),
        ],
        out_specs=[
            pl.BlockSpec((_BR, _D), lambda i: (i, 0)),
            pl.BlockSpec((_BR, _D), lambda i: (i, 0)),
        ],
        out_shape=[_SDS((_N, _D), jnp.float32)] * 2,
    )(xa, wa, dega3, xf, wf, degf3)


def _t2_body(a1_ref, a2_ref, dega_ref, b1aa_ref, b1ab_ref,
             f1_ref, f2_ref, degf_ref, b1fa_ref, b1fb_ref,
             wl1a_ref, wl1b_ref, bl1_ref,
             enca_ref, encf_ref, esuma_ref, esumf_ref):
    i = pl.program_id(0)
    sa = lax.rsqrt(jnp.maximum(dega_ref[0, 0, :], 1.0))
    ha1 = jnp.maximum(a1_ref[...] * sa[:, None] + b1aa_ref[...], 0.0)
    ha2 = jnp.maximum(a2_ref[...] * sa[:, None] + b1ab_ref[...], 0.0)
    ea = (jnp.dot(ha1, wl1a_ref[...], preferred_element_type=jnp.float32)
          + jnp.dot(ha2, wl1b_ref[...], preferred_element_type=jnp.float32)
          + bl1_ref[...])
    enca_ref[...] = ea
    sf = lax.rsqrt(jnp.maximum(degf_ref[0, 0, :], 1.0))
    hf1 = jnp.maximum(f1_ref[...] * sf[:, None] + b1fa_ref[...], 0.0)
    hf2 = jnp.maximum(f2_ref[...] * sf[:, None] + b1fb_ref[...], 0.0)
    ef = (jnp.dot(hf1, wl1a_ref[...], preferred_element_type=jnp.float32)
          + jnp.dot(hf2, wl1b_ref[...], preferred_element_type=jnp.float32)
          + bl1_ref[...])
    encf_ref[...] = ef

    @pl.when(i == 0)
    def _():
        esuma_ref[...] = jnp.zeros_like(esuma_ref)
        esumf_ref[...] = jnp.zeros_like(esumf_ref)

    esuma_ref[...] += jnp.sum(ea, axis=0, keepdims=True)
    esumf_ref[...] += jnp.sum(ef, axis=0, keepdims=True)


def _t2(a1, a2, dega3, b1a, f1, f2, degf3, b1f, wl1, bl1):
    half = pl.BlockSpec((_BR, _HD), lambda i: (i, 0))
    deg = pl.BlockSpec((1, 1, _BR), lambda i: (i, 0, 0))
    vhd = pl.BlockSpec((1, _HD), lambda i: (0, 0))
    wsp = pl.BlockSpec((_HD, _U), lambda i: (0, 0))
    vu = pl.BlockSpec((1, _U), lambda i: (0, 0))
    return pl.pallas_call(
        _t2_body,
        grid=(_NB,),
        in_specs=[half, half, deg, vhd, vhd,
                  half, half, deg, vhd, vhd,
                  wsp, wsp, vu],
        out_specs=[
            pl.BlockSpec((_BR, _U), lambda i: (i, 0)),
            pl.BlockSpec((_BR, _U), lambda i: (i, 0)),
            vu, vu,
        ],
        out_shape=[_SDS((_N, _U), jnp.float32), _SDS((_N, _U), jnp.float32),
                   _SDS((1, _U), jnp.float32), _SDS((1, _U), jnp.float32)],
    )(a1, a2, dega3, b1a[:, :_HD], b1a[:, _HD:],
      f1, f2, degf3, b1f[:, :_HD], b1f[:, _HD:],
      wl1[:_HD], wl1[_HD:], bl1)


def _t3_body(enca_ref, sumf_ref, encf_ref, suma_ref, wl2_ref, bl2_ref,
             wa2_ref, wf2_ref, dega_ref, degf_ref, oa_ref, of_ref):
    ea = enca_ref[...] + 0.1 * sumf_ref[...]
    deca = jnp.dot(ea, wl2_ref[...], preferred_element_type=jnp.float32) \
        + bl2_ref[...]
    xwa = jnp.dot(deca, wa2_ref[...], preferred_element_type=jnp.float32)
    sa = lax.rsqrt(jnp.maximum(dega_ref[0, 0, :], 1.0))
    oa_ref[...] = xwa * sa[:, None]
    ef = encf_ref[...] + 0.1 * suma_ref[...]
    decf = jnp.dot(ef, wl2_ref[...], preferred_element_type=jnp.float32) \
        + bl2_ref[...]
    xwf = jnp.dot(decf, wf2_ref[...], preferred_element_type=jnp.float32)
    sf = lax.rsqrt(jnp.maximum(degf_ref[0, 0, :], 1.0))
    of_ref[...] = xwf * sf[:, None]


def _t3(enca, sumf, encf, suma, wl2, bl2, wa2, wf2, dega3, degf3):
    enc = pl.BlockSpec((_BR, _U), lambda i: (i, 0))
    vu = pl.BlockSpec((1, _U), lambda i: (0, 0))
    wl2s = pl.BlockSpec((_U, _D), lambda i: (0, 0))
    vd = pl.BlockSpec((1, _D), lambda i: (0, 0))
    wsq = pl.BlockSpec((_D, _D), lambda i: (0, 0))
    deg = pl.BlockSpec((1, 1, _BR), lambda i: (i, 0, 0))
    outs = pl.BlockSpec((_BR, _D), lambda i: (i, 0))
    return pl.pallas_call(
        _t3_body,
        grid=(_NB,),
        in_specs=[enc, vu, enc, vu, wl2s, vd, wsq, wsq, deg, deg],
        out_specs=[outs, outs],
        out_shape=[_SDS((_N, _D), jnp.float32)] * 2,
    )(enca, sumf, encf, suma, wl2, bl2, wa2, wf2, dega3, degf3)


def _t4_body(aa1_ref, aa2_ref, dega_ref, b2aa_ref, b2ab_ref,
             af1_ref, af2_ref, degf_ref, b2fa_ref, b2fb_ref,
             wca_ref, wcb_ref, bc_ref, o_ref,
             asum1_ref, asum2_ref, fmax1_ref, fmax2_ref):
    i = pl.program_id(0)
    sa = lax.rsqrt(jnp.maximum(dega_ref[0, 0, :], 1.0))
    ha1 = jnp.maximum(aa1_ref[...] * sa[:, None] + b2aa_ref[...], 0.0)
    ha2 = jnp.maximum(aa2_ref[...] * sa[:, None] + b2ab_ref[...], 0.0)
    sf = lax.rsqrt(jnp.maximum(degf_ref[0, 0, :], 1.0))
    hf1 = jnp.maximum(af1_ref[...] * sf[:, None] + b2fa_ref[...], 0.0)
    hf2 = jnp.maximum(af2_ref[...] * sf[:, None] + b2fb_ref[...], 0.0)

    @pl.when(i == 0)
    def _():
        asum1_ref[...] = jnp.zeros_like(asum1_ref)
        asum2_ref[...] = jnp.zeros_like(asum2_ref)
        fmax1_ref[...] = jnp.full_like(fmax1_ref, -jnp.inf)
        fmax2_ref[...] = jnp.full_like(fmax2_ref, -jnp.inf)

    asum1_ref[...] += jnp.sum(ha1, axis=0, keepdims=True)
    asum2_ref[...] += jnp.sum(ha2, axis=0, keepdims=True)
    fmax1_ref[...] = jnp.maximum(fmax1_ref[...],
                                 jnp.max(hf1, axis=0, keepdims=True))
    fmax2_ref[...] = jnp.maximum(fmax2_ref[...],
                                 jnp.max(hf2, axis=0, keepdims=True))

    @pl.when(i == _NB - 1)
    def _():
        def norm2(v1, v2):
            m = (jnp.sum(v1) + jnp.sum(v2)) / _D
            ss = jnp.sum((v1 - m) ** 2) + jnp.sum((v2 - m) ** 2)
            sd = jnp.sqrt(ss / (_D - 1))
            w1 = (v1 - m) / sd
            w2 = (v2 - m) / sd
            mn = jnp.minimum(jnp.min(w1), jnp.min(w2))
            mx = jnp.maximum(jnp.max(w1), jnp.max(w2))
            return (w1 - mn) / (mx - mn), (w2 - mn) / (mx - mn)

        na1, na2 = norm2(asum1_ref[...] / _N, asum2_ref[...] / _N)
        nf1, nf2 = norm2(fmax1_ref[...], fmax2_ref[...])
        e1 = na1 + nf1
        e2 = na2 + nf2
        o_ref[...] = (jnp.dot(e1, wca_ref[...],
                              preferred_element_type=jnp.float32)
                      + jnp.dot(e2, wcb_ref[...],
                                preferred_element_type=jnp.float32)
                      + bc_ref[...])


def _t4(aa1, aa2, dega3, b2a, af1, af2, degf3, b2f, wc, bc):
    half = pl.BlockSpec((_BR, _HD), lambda i: (i, 0))
    deg = pl.BlockSpec((1, 1, _BR), lambda i: (i, 0, 0))
    vhd = pl.BlockSpec((1, _HD), lambda i: (0, 0))
    return pl.pallas_call(
        _t4_body,
        grid=(_NB,),
        in_specs=[half, half, deg, vhd, vhd,
                  half, half, deg, vhd, vhd,
                  pl.BlockSpec((_HD, _D), lambda i: (0, 0)),
                  pl.BlockSpec((_HD, _D), lambda i: (0, 0)),
                  pl.BlockSpec((1, _D), lambda i: (0, 0))],
        out_specs=pl.BlockSpec((1, _D), lambda i: (0, 0)),
        out_shape=_SDS((1, _D), jnp.float32),
        scratch_shapes=[
            pltpu.VMEM((1, _HD), jnp.float32),
            pltpu.VMEM((1, _HD), jnp.float32),
            pltpu.VMEM((1, _HD), jnp.float32),
            pltpu.VMEM((1, _HD), jnp.float32),
        ],
    )(aa1, aa2, dega3, b2a[:, :_HD], b2a[:, _HD:],
      af1, af2, degf3, b2f[:, :_HD], b2f[:, _HD:],
      wc[:_HD], wc[_HD:], bc)


def kernel(apig, apig_feat, fcg, fcg_feat,
           W_a1, b_a1, W_a2, b_a2, W_f1, b_f1, W_f2, b_f2,
           W_l1, b_l1, W_l2, b_l2, W_attn, b_attn, W_c, b_c):
    f32 = jnp.float32
    src_a = apig[0].reshape(16, _CPT, _CH)
    dst_a = apig[1].reshape(16, _CPT, _CH)
    src_f = fcg[0].reshape(16, _CPT, _CH)
    dst_f = fcg[1].reshape(16, _CPT, _CH)

    d_as, d_ad, d_fs, d_fd = _deg_call(src_a, dst_a, src_f, dst_f)

    def d3(d):
        return d[:_N].reshape(_NB, 1, _BR)

    d_as3, d_ad3, d_fs3, d_fd3 = d3(d_as), d3(d_ad), d3(d_fs), d3(d_fd)

    xws_a, xws_f = _t1(apig_feat, W_a1, d_as3, fcg_feat, W_f1, d_fs3)
    agg_a1, agg_a2, agg_f1, agg_f2 = _agg_call(
        xws_a[:, :_HD], xws_a[:, _HD:], xws_f[:, :_HD], xws_f[:, _HD:],
        src_a, dst_a, src_f, dst_f)

    enc_a, enc_f, esum_a, esum_f = _t2(
        agg_a1[:_N], agg_a2[:_N], d_ad3, b_a1.reshape(1, _D),
        agg_f1[:_N], agg_f2[:_N], d_fd3, b_f1.reshape(1, _D),
        W_l1, b_l1.reshape(1, _U))

    xws2_a, xws2_f = _t3(enc_a, esum_f, enc_f, esum_a,
                         W_l2, b_l2.reshape(1, _D), W_a2, W_f2,
                         d_as3, d_fs3)
    agg2_a1, agg2_a2, agg2_f1, agg2_f2 = _agg_call(
        xws2_a[:, :_HD], xws2_a[:, _HD:], xws2_f[:, :_HD], xws2_f[:, _HD:],
        src_a, dst_a, src_f, dst_f)

    wc_pad = jnp.zeros((_D, _D), f32).at[:, :10].set(W_c)
    bc_pad = jnp.zeros((1, _D), f32).at[0, :10].set(b_c)
    out = _t4(agg2_a1[:_N], agg2_a2[:_N], d_ad3, b_a2.reshape(1, _D),
              agg2_f1[:_N], agg2_f2[:_N], d_fd3, b_f2.reshape(1, _D),
              wc_pad, bc_pad)
    return out[0, :10]


# speedup vs baseline: 15.2481x; 1.0428x over previous
import functools

import jax
import jax.numpy as jnp
from jax import lax
from jax.experimental import pallas as pl
from jax.experimental.pallas import tpu as pltpu
from jax.experimental.pallas import tpu_sc as plsc

_N = 10000
_E = 320000
_D = 128
_HD = _D // 2
_U = 64
_CH = 80
_CPT = _E // 16 // _CH
_NPAD = 10240
_RPT = _NPAD // 16
_NB = 10
_BR = _N // _NB
_NB2 = 16
_BR2 = _NPAD // _NB2
_NBUF = 5

_SDS = jax.ShapeDtypeStruct
_mesh = plsc.VectorSubcoreMesh(core_axis_name="c", subcore_axis_name="s")


_DPT = _NPAD // 16


@functools.partial(
    pl.kernel,
    mesh=_mesh,
    out_type=[_SDS((_NPAD,), jnp.float32)] * 4,
    scratch_types=[
        pltpu.VMEM((_CPT, _CH), jnp.int32),
        pltpu.VMEM((_CPT, _CH), jnp.int32),
        pltpu.VMEM((_NPAD,), jnp.float32),
        pltpu.VMEM((_NPAD,), jnp.float32),
        pltpu.VMEM((16, _DPT), jnp.float32),
        pltpu.VMEM((_DPT,), jnp.float32),
        pltpu.VMEM_SHARED((16, _NPAD), jnp.float32),
        pltpu.VMEM_SHARED((16, _NPAD), jnp.float32),
    ],
    compiler_params=pltpu.CompilerParams(needs_layout_passes=False),
)
def _deg_call(src_a, dst_a, src_f, dst_f,
              out_as, out_ad, out_fs, out_fd,
              srcv, dstv, degs_v, degd_v, red_v, outb_v, sh_s, sh_d):
    c = lax.axis_index("c")
    s = lax.axis_index("s")

    def run(src3d, dst3d, out_s, out_d):
        pltpu.sync_copy(src3d.at[s], srcv)
        pltpu.sync_copy(dst3d.at[s], dstv)
        z16 = jnp.zeros((16,), jnp.float32)

        def zbody(i, carry):
            degs_v[pl.ds(i * 16, 16)] = z16
            degd_v[pl.ds(i * 16, 16)] = z16
            return carry

        lax.fori_loop(0, _NPAD // 16, zbody, 0)
        ones = jnp.ones((16,), jnp.float32)

        def body(r, carry):
            for cc in range(_CH // 16):
                si = srcv[r, pl.ds(cc * 16, 16)]
                di = dstv[r, pl.ds(cc * 16, 16)]
                plsc.addupdate_scatter(degs_v, [si], ones)
                plsc.addupdate_scatter(degd_v, [di], ones)
            return carry

        lax.fori_loop(0, _CPT, body, 0)
        pltpu.sync_copy(degs_v, sh_s.at[s])
        pltpu.sync_copy(degd_v, sh_d.at[s])
        plsc.subcore_barrier()

        def reduce_out(sh, out):
            pltpu.sync_copy(sh.at[:, pl.ds(s * _DPT, _DPT)], red_v)
            for chk in range(_DPT // 16):
                v = red_v[0, pl.ds(chk * 16, 16)]
                for t in range(1, 16):
                    v = v + red_v[t, pl.ds(chk * 16, 16)]
                outb_v[pl.ds(chk * 16, 16)] = v
            pltpu.sync_copy(outb_v, out.at[pl.ds(s * _DPT, _DPT)])

        reduce_out(sh_s, out_s)
        reduce_out(sh_d, out_d)

    @pl.when(c == 0)
    def _():
        run(src_a, dst_a, out_as, out_ad)

    @pl.when(c == 1)
    def _():
        run(src_f, dst_f, out_fs, out_fd)


@functools.partial(
    pl.kernel,
    mesh=_mesh,
    out_type=[_SDS((_NPAD, _HD), jnp.float32)] * 4,
    scratch_types=[
        pltpu.VMEM((_CPT, _CH), jnp.int32),
        pltpu.VMEM((_CPT, _CH), jnp.int32),
        pltpu.VMEM((_CH, _HD), jnp.float32),
    ] + [pltpu.VMEM((_CH, _HD), jnp.float32) for _ in range(_NBUF)]
      + [pltpu.SemaphoreType.DMA for _ in range(2 * _NBUF)]
      + [pltpu.VMEM_SHARED((_NPAD, _HD), jnp.float32)],
    compiler_params=pltpu.CompilerParams(use_tc_tiling_on_sc=False),
)
def _agg_call(xa1, xa2, xf1, xf2, src_a, dst_a, src_f, dst_f,
              out_a1, out_a2, out_f1, out_f2, *scr):
    srcv, dstv, zbuf = scr[0], scr[1], scr[2]
    rows = list(scr[3:3 + _NBUF])
    gsem = list(scr[3 + _NBUF:3 + 2 * _NBUF])
    ssem = list(scr[3 + 2 * _NBUF:3 + 3 * _NBUF])
    acc = scr[3 + 3 * _NBUF]
    c = lax.axis_index("c")
    s = lax.axis_index("s")

    def zero_acc():
        for k in range(_RPT // _CH):
            pltpu.sync_copy(zbuf, acc.at[pl.ds(s * _RPT + k * _CH, _CH)])

    def pipeline(xws):
        pltpu.async_copy(xws.at[srcv.at[0]], rows[0], gsem[0])
        pltpu.async_copy(xws.at[srcv.at[1]], rows[1], gsem[1])

        def body(i, carry):
            for b in range(_NBUF):
                j = _NBUF * i + b
                b2 = (b + 2) % _NBUF

                @pl.when((j >= 3) & (j + 2 < _CPT))
                def _():
                    pltpu.make_async_copy(
                        rows[b2], acc.at[dstv.at[j]], ssem[b2]).wait()

                @pl.when(j + 2 < _CPT)
                def _():
                    pltpu.async_copy(
                        xws.at[srcv.at[j + 2]], rows[b2], gsem[b2])

                pltpu.make_async_copy(
                    xws.at[srcv.at[j]], rows[b], gsem[b]).wait()
                pltpu.async_copy(
                    rows[b], acc.at[dstv.at[j]], ssem[b], add=True)
            return carry

        lax.fori_loop(0, _CPT // _NBUF, body, 0)
        for b in range(_NBUF):
            pltpu.make_async_copy(
                rows[b], acc.at[dstv.at[0]], ssem[b]).wait()

    def run(xws1, xws2, src3d, dst3d, out1, out2):
        pltpu.sync_copy(src3d.at[s], srcv)
        pltpu.sync_copy(dst3d.at[s], dstv)
        z16 = jnp.zeros((16,), jnp.float32)

        def zbody(r, carry):
            for cc in range(_HD // 16):
                zbuf[r, pl.ds(cc * 16, 16)] = z16
            return carry

        lax.fori_loop(0, _CH, zbody, 0)
        zero_acc()
        plsc.subcore_barrier()
        pipeline(xws1)
        plsc.subcore_barrier()
        pltpu.sync_copy(acc.at[pl.ds(s * _RPT, _RPT)],
                        out1.at[pl.ds(s * _RPT, _RPT)])
        zero_acc()
        plsc.subcore_barrier()
        pipeline(xws2)
        plsc.subcore_barrier()
        pltpu.sync_copy(acc.at[pl.ds(s * _RPT, _RPT)],
                        out2.at[pl.ds(s * _RPT, _RPT)])

    @pl.when(c == 0)
    def _():
        run(xa1, xa2, src_a, dst_a, out_a1, out_a2)

    @pl.when(c == 1)
    def _():
        run(xf1, xf2, src_f, dst_f, out_f1, out_f2)


def _t1_body(xa_ref, wa_ref, dega_ref, xf_ref, wf_ref, degf_ref,
             oa1_ref, oa2_ref, of1_ref, of2_ref):
    sa = lax.rsqrt(jnp.maximum(dega_ref[0, 0, :], 1.0))
    xwa = jnp.dot(xa_ref[...], wa_ref[...], preferred_element_type=jnp.float32)
    xwa = xwa * sa[:, None]
    oa1_ref[...] = xwa[:, :_HD]
    oa2_ref[...] = xwa[:, _HD:]
    sf = lax.rsqrt(jnp.maximum(degf_ref[0, 0, :], 1.0))
    xwf = jnp.dot(xf_ref[...], wf_ref[...], preferred_element_type=jnp.float32)
    xwf = xwf * sf[:, None]
    of1_ref[...] = xwf[:, :_HD]
    of2_ref[...] = xwf[:, _HD:]


def _t1(xa, wa, dega3, xf, wf, degf3):
    return pl.pallas_call(
        _t1_body,
        grid=(_NB,),
        in_specs=[
            pl.BlockSpec((_BR, _D), lambda i: (i, 0)),
            pl.BlockSpec((_D, _D), lambda i: (0, 0)),
            pl.BlockSpec((1, 1, _BR), lambda i: (i, 0, 0)),
            pl.BlockSpec((_BR, _D), lambda i: (i, 0)),
            pl.BlockSpec((_D, _D), lambda i: (0, 0)),
            pl.BlockSpec((1, 1, _BR), lambda i: (i, 0, 0)),
        ],
        out_specs=[pl.BlockSpec((_BR, _HD), lambda i: (i, 0))] * 4,
        out_shape=[_SDS((_N, _HD), jnp.float32)] * 4,
    )(xa, wa, dega3, xf, wf, degf3)


def _t2_body(a1_ref, a2_ref, dega_ref, b1aa_ref, b1ab_ref,
             f1_ref, f2_ref, degf_ref, b1fa_ref, b1fb_ref,
             wl1a_ref, wl1b_ref, bl1_ref,
             enca_ref, encf_ref, esuma_ref, esumf_ref):
    i = pl.program_id(0)
    row = i * _BR2 + lax.broadcasted_iota(jnp.int32, (_BR2, 1), 0)
    valid = row < _N
    sa = lax.rsqrt(jnp.maximum(dega_ref[0, 0, :], 1.0))
    ha1 = jnp.maximum(a1_ref[...] * sa[:, None] + b1aa_ref[...], 0.0)
    ha2 = jnp.maximum(a2_ref[...] * sa[:, None] + b1ab_ref[...], 0.0)
    ea = (jnp.dot(ha1, wl1a_ref[...], preferred_element_type=jnp.float32)
          + jnp.dot(ha2, wl1b_ref[...], preferred_element_type=jnp.float32)
          + bl1_ref[...])
    enca_ref[...] = ea
    sf = lax.rsqrt(jnp.maximum(degf_ref[0, 0, :], 1.0))
    hf1 = jnp.maximum(f1_ref[...] * sf[:, None] + b1fa_ref[...], 0.0)
    hf2 = jnp.maximum(f2_ref[...] * sf[:, None] + b1fb_ref[...], 0.0)
    ef = (jnp.dot(hf1, wl1a_ref[...], preferred_element_type=jnp.float32)
          + jnp.dot(hf2, wl1b_ref[...], preferred_element_type=jnp.float32)
          + bl1_ref[...])
    encf_ref[...] = ef

    @pl.when(i == 0)
    def _():
        esuma_ref[...] = jnp.zeros_like(esuma_ref)
        esumf_ref[...] = jnp.zeros_like(esumf_ref)

    zero = jnp.zeros_like(ea)
    esuma_ref[...] += jnp.sum(jnp.where(valid, ea, zero), axis=0,
                              keepdims=True)
    esumf_ref[...] += jnp.sum(jnp.where(valid, ef, zero), axis=0,
                              keepdims=True)


def _t2(a1, a2, dega3, b1a, f1, f2, degf3, b1f, wl1, bl1):
    half = pl.BlockSpec((_BR2, _HD), lambda i: (i, 0))
    deg = pl.BlockSpec((1, 1, _BR2), lambda i: (i, 0, 0))
    vhd = pl.BlockSpec((1, _HD), lambda i: (0, 0))
    wsp = pl.BlockSpec((_HD, _U), lambda i: (0, 0))
    vu = pl.BlockSpec((1, _U), lambda i: (0, 0))
    return pl.pallas_call(
        _t2_body,
        grid=(_NB2,),
        in_specs=[half, half, deg, vhd, vhd,
                  half, half, deg, vhd, vhd,
                  wsp, wsp, vu],
        out_specs=[
            pl.BlockSpec((_BR2, _U), lambda i: (i, 0)),
            pl.BlockSpec((_BR2, _U), lambda i: (i, 0)),
            vu, vu,
        ],
        out_shape=[_SDS((_NPAD, _U), jnp.float32),
                   _SDS((_NPAD, _U), jnp.float32),
                   _SDS((1, _U), jnp.float32), _SDS((1, _U), jnp.float32)],
    )(a1, a2, dega3, b1a[:, :_HD], b1a[:, _HD:],
      f1, f2, degf3, b1f[:, :_HD], b1f[:, _HD:],
      wl1[:_HD], wl1[_HD:], bl1)


def _t3_body(enca_ref, sumf_ref, encf_ref, suma_ref, wl2_ref, bl2_ref,
             wa2_ref, wf2_ref, dega_ref, degf_ref,
             oa1_ref, oa2_ref, of1_ref, of2_ref):
    ea = enca_ref[...] + 0.1 * sumf_ref[...]
    deca = jnp.dot(ea, wl2_ref[...], preferred_element_type=jnp.float32) \
        + bl2_ref[...]
    xwa = jnp.dot(deca, wa2_ref[...], preferred_element_type=jnp.float32)
    sa = lax.rsqrt(jnp.maximum(dega_ref[0, 0, :], 1.0))
    xwa = xwa * sa[:, None]
    oa1_ref[...] = xwa[:, :_HD]
    oa2_ref[...] = xwa[:, _HD:]
    ef = encf_ref[...] + 0.1 * suma_ref[...]
    decf = jnp.dot(ef, wl2_ref[...], preferred_element_type=jnp.float32) \
        + bl2_ref[...]
    xwf = jnp.dot(decf, wf2_ref[...], preferred_element_type=jnp.float32)
    sf = lax.rsqrt(jnp.maximum(degf_ref[0, 0, :], 1.0))
    xwf = xwf * sf[:, None]
    of1_ref[...] = xwf[:, :_HD]
    of2_ref[...] = xwf[:, _HD:]


def _t3(enca, sumf, encf, suma, wl2, bl2, wa2, wf2, dega3, degf3):
    enc = pl.BlockSpec((_BR2, _U), lambda i: (i, 0))
    vu = pl.BlockSpec((1, _U), lambda i: (0, 0))
    wl2s = pl.BlockSpec((_U, _D), lambda i: (0, 0))
    vd = pl.BlockSpec((1, _D), lambda i: (0, 0))
    wsq = pl.BlockSpec((_D, _D), lambda i: (0, 0))
    deg = pl.BlockSpec((1, 1, _BR2), lambda i: (i, 0, 0))
    outs = pl.BlockSpec((_BR2, _HD), lambda i: (i, 0))
    return pl.pallas_call(
        _t3_body,
        grid=(_NB2,),
        in_specs=[enc, vu, enc, vu, wl2s, vd, wsq, wsq, deg, deg],
        out_specs=[outs] * 4,
        out_shape=[_SDS((_NPAD, _HD), jnp.float32)] * 4,
    )(enca, sumf, encf, suma, wl2, bl2, wa2, wf2, dega3, degf3)


def _t4_body(aa1_ref, aa2_ref, dega_ref, b2aa_ref, b2ab_ref,
             af1_ref, af2_ref, degf_ref, b2fa_ref, b2fb_ref,
             wca_ref, wcb_ref, bc_ref, o_ref,
             asum1_ref, asum2_ref, fmax1_ref, fmax2_ref):
    i = pl.program_id(0)
    row = i * _BR2 + lax.broadcasted_iota(jnp.int32, (_BR2, 1), 0)
    valid = row < _N
    ninf = jnp.float32(-jnp.inf)
    sa = lax.rsqrt(jnp.maximum(dega_ref[0, 0, :], 1.0))
    ha1 = jnp.maximum(aa1_ref[...] * sa[:, None] + b2aa_ref[...], 0.0)
    ha2 = jnp.maximum(aa2_ref[...] * sa[:, None] + b2ab_ref[...], 0.0)
    sf = lax.rsqrt(jnp.maximum(degf_ref[0, 0, :], 1.0))
    hf1 = jnp.maximum(af1_ref[...] * sf[:, None] + b2fa_ref[...], 0.0)
    hf2 = jnp.maximum(af2_ref[...] * sf[:, None] + b2fb_ref[...], 0.0)
    zero = jnp.zeros_like(ha1)
    ha1 = jnp.where(valid, ha1, zero)
    ha2 = jnp.where(valid, ha2, zero)
    hf1 = jnp.where(valid, hf1, ninf)
    hf2 = jnp.where(valid, hf2, ninf)

    @pl.when(i == 0)
    def _():
        asum1_ref[...] = jnp.zeros_like(asum1_ref)
        asum2_ref[...] = jnp.zeros_like(asum2_ref)
        fmax1_ref[...] = jnp.full_like(fmax1_ref, -jnp.inf)
        fmax2_ref[...] = jnp.full_like(fmax2_ref, -jnp.inf)

    asum1_ref[...] += jnp.sum(ha1, axis=0, keepdims=True)
    asum2_ref[...] += jnp.sum(ha2, axis=0, keepdims=True)
    fmax1_ref[...] = jnp.maximum(fmax1_ref[...],
                                 jnp.max(hf1, axis=0, keepdims=True))
    fmax2_ref[...] = jnp.maximum(fmax2_ref[...],
                                 jnp.max(hf2, axis=0, keepdims=True))

    @pl.when(i == _NB2 - 1)
    def _():
        def norm2(v1, v2):
            m = (jnp.sum(v1) + jnp.sum(v2)) / _D
            ss = jnp.sum((v1 - m) ** 2) + jnp.sum((v2 - m) ** 2)
            sd = jnp.sqrt(ss / (_D - 1))
            w1 = (v1 - m) / sd
            w2 = (v2 - m) / sd
            mn = jnp.minimum(jnp.min(w1), jnp.min(w2))
            mx = jnp.maximum(jnp.max(w1), jnp.max(w2))
            return (w1 - mn) / (mx - mn), (w2 - mn) / (mx - mn)

        na1, na2 = norm2(asum1_ref[...] / _N, asum2_ref[...] / _N)
        nf1, nf2 = norm2(fmax1_ref[...], fmax2_ref[...])
        e1 = na1 + nf1
        e2 = na2 + nf2
        o_ref[...] = (jnp.dot(e1, wca_ref[...],
                              preferred_element_type=jnp.float32)
                      + jnp.dot(e2, wcb_ref[...],
                                preferred_element_type=jnp.float32)
                      + bc_ref[...])


def _t4(aa1, aa2, dega3, b2a, af1, af2, degf3, b2f, wc, bc):
    half = pl.BlockSpec((_BR2, _HD), lambda i: (i, 0))
    deg = pl.BlockSpec((1, 1, _BR2), lambda i: (i, 0, 0))
    vhd = pl.BlockSpec((1, _HD), lambda i: (0, 0))
    return pl.pallas_call(
        _t4_body,
        grid=(_NB2,),
        in_specs=[half, half, deg, vhd, vhd,
                  half, half, deg, vhd, vhd,
                  pl.BlockSpec((_HD, _D), lambda i: (0, 0)),
                  pl.BlockSpec((_HD, _D), lambda i: (0, 0)),
                  pl.BlockSpec((1, _D), lambda i: (0, 0))],
        out_specs=pl.BlockSpec((1, _D), lambda i: (0, 0)),
        out_shape=_SDS((1, _D), jnp.float32),
        scratch_shapes=[
            pltpu.VMEM((1, _HD), jnp.float32),
            pltpu.VMEM((1, _HD), jnp.float32),
            pltpu.VMEM((1, _HD), jnp.float32),
            pltpu.VMEM((1, _HD), jnp.float32),
        ],
    )(aa1, aa2, dega3, b2a[:, :_HD], b2a[:, _HD:],
      af1, af2, degf3, b2f[:, :_HD], b2f[:, _HD:],
      wc[:_HD], wc[_HD:], bc)


def kernel(apig, apig_feat, fcg, fcg_feat,
           W_a1, b_a1, W_a2, b_a2, W_f1, b_f1, W_f2, b_f2,
           W_l1, b_l1, W_l2, b_l2, W_attn, b_attn, W_c, b_c):
    f32 = jnp.float32
    src_a = apig[0].reshape(16, _CPT, _CH)
    dst_a = apig[1].reshape(16, _CPT, _CH)
    src_f = fcg[0].reshape(16, _CPT, _CH)
    dst_f = fcg[1].reshape(16, _CPT, _CH)

    d_as, d_ad, d_fs, d_fd = _deg_call(src_a, dst_a, src_f, dst_f)

    def d3(d):
        return d[:_N].reshape(_NB, 1, _BR)

    def d3p(d):
        return d.reshape(_NB2, 1, _BR2)

    xws_a1, xws_a2, xws_f1, xws_f2 = _t1(apig_feat, W_a1, d3(d_as),
                                         fcg_feat, W_f1, d3(d_fs))
    agg_a1, agg_a2, agg_f1, agg_f2 = _agg_call(
        xws_a1, xws_a2, xws_f1, xws_f2, src_a, dst_a, src_f, dst_f)

    enc_a, enc_f, esum_a, esum_f = _t2(
        agg_a1, agg_a2, d3p(d_ad), b_a1.reshape(1, _D),
        agg_f1, agg_f2, d3p(d_fd), b_f1.reshape(1, _D),
        W_l1, b_l1.reshape(1, _U))

    xws2_a1, xws2_a2, xws2_f1, xws2_f2 = _t3(
        enc_a, esum_f, enc_f, esum_a,
        W_l2, b_l2.reshape(1, _D), W_a2, W_f2, d3p(d_as), d3p(d_fs))
    agg2_a1, agg2_a2, agg2_f1, agg2_f2 = _agg_call(
        xws2_a1, xws2_a2, xws2_f1, xws2_f2, src_a, dst_a, src_f, dst_f)

    wc_pad = jnp.zeros((_D, _D), f32).at[:, :10].set(W_c)
    bc_pad = jnp.zeros((1, _D), f32).at[0, :10].set(b_c)
    out = _t4(agg2_a1, agg2_a2, d3p(d_ad), b_a2.reshape(1, _D),
              agg2_f1, agg2_f2, d3p(d_fd), b_f2.reshape(1, _D),
              wc_pad, bc_pad)
    return out[0, :10]


# speedup vs baseline: 16.2994x; 1.0689x over previous
import functools

import jax
import jax.numpy as jnp
from jax import lax
from jax.experimental import pallas as pl
from jax.experimental.pallas import tpu as pltpu
from jax.experimental.pallas import tpu_sc as plsc

_N = 10000
_E = 320000
_D = 128
_HD = _D // 2
_U = 64
_CH = 80
_CPT = _E // 16 // _CH
_NPAD = 10240
_RPT = _NPAD // 16
_NB = 10
_BR = _N // _NB
_NB2 = 16
_BR2 = _NPAD // _NB2
_NBUF = 5
_LAG = 3

_SDS = jax.ShapeDtypeStruct
_mesh = plsc.VectorSubcoreMesh(core_axis_name="c", subcore_axis_name="s")


_DPT = _NPAD // 16


@functools.partial(
    pl.kernel,
    mesh=_mesh,
    out_type=[_SDS((_NPAD,), jnp.float32)] * 4,
    scratch_types=[
        pltpu.VMEM((_CPT, _CH), jnp.int32),
        pltpu.VMEM((_CPT, _CH), jnp.int32),
        pltpu.VMEM((_NPAD,), jnp.float32),
        pltpu.VMEM((_NPAD,), jnp.float32),
        pltpu.VMEM((16, _DPT), jnp.float32),
        pltpu.VMEM((_DPT,), jnp.float32),
        pltpu.VMEM_SHARED((16, _NPAD), jnp.float32),
        pltpu.VMEM_SHARED((16, _NPAD), jnp.float32),
    ],
    compiler_params=pltpu.CompilerParams(needs_layout_passes=False),
)
def _deg_call(src_a, dst_a, src_f, dst_f,
              out_as, out_ad, out_fs, out_fd,
              srcv, dstv, degs_v, degd_v, red_v, outb_v, sh_s, sh_d):
    c = lax.axis_index("c")
    s = lax.axis_index("s")

    def run(src3d, dst3d, out_s, out_d):
        pltpu.sync_copy(src3d.at[s], srcv)
        pltpu.sync_copy(dst3d.at[s], dstv)
        z16 = jnp.zeros((16,), jnp.float32)

        def zbody(i, carry):
            degs_v[pl.ds(i * 16, 16)] = z16
            degd_v[pl.ds(i * 16, 16)] = z16
            return carry

        lax.fori_loop(0, _NPAD // 16, zbody, 0)
        ones = jnp.ones((16,), jnp.float32)

        def body(r, carry):
            for cc in range(_CH // 16):
                si = srcv[r, pl.ds(cc * 16, 16)]
                di = dstv[r, pl.ds(cc * 16, 16)]
                plsc.addupdate_scatter(degs_v, [si], ones)
                plsc.addupdate_scatter(degd_v, [di], ones)
            return carry

        lax.fori_loop(0, _CPT, body, 0)
        pltpu.sync_copy(degs_v, sh_s.at[s])
        pltpu.sync_copy(degd_v, sh_d.at[s])
        plsc.subcore_barrier()

        def reduce_out(sh, out):
            pltpu.sync_copy(sh.at[:, pl.ds(s * _DPT, _DPT)], red_v)
            for chk in range(_DPT // 16):
                v = red_v[0, pl.ds(chk * 16, 16)]
                for t in range(1, 16):
                    v = v + red_v[t, pl.ds(chk * 16, 16)]
                outb_v[pl.ds(chk * 16, 16)] = v
            pltpu.sync_copy(outb_v, out.at[pl.ds(s * _DPT, _DPT)])

        reduce_out(sh_s, out_s)
        reduce_out(sh_d, out_d)

    @pl.when(c == 0)
    def _():
        run(src_a, dst_a, out_as, out_ad)

    @pl.when(c == 1)
    def _():
        run(src_f, dst_f, out_fs, out_fd)


@functools.partial(
    pl.kernel,
    mesh=_mesh,
    out_type=[_SDS((_NPAD, _HD), jnp.float32)] * 4,
    scratch_types=[
        pltpu.VMEM((_CPT, _CH), jnp.int32),
        pltpu.VMEM((_CPT, _CH), jnp.int32),
        pltpu.VMEM((_CH, _HD), jnp.float32),
    ] + [pltpu.VMEM((_CH, _HD), jnp.float32) for _ in range(_NBUF)]
      + [pltpu.SemaphoreType.DMA for _ in range(2 * _NBUF)]
      + [pltpu.VMEM_SHARED((_NPAD, _HD), jnp.float32)],
    compiler_params=pltpu.CompilerParams(use_tc_tiling_on_sc=False),
)
def _agg_call(xa1, xa2, xf1, xf2, src_a, dst_a, src_f, dst_f,
              out_a1, out_a2, out_f1, out_f2, *scr):
    srcv, dstv, zbuf = scr[0], scr[1], scr[2]
    rows = list(scr[3:3 + _NBUF])
    gsem = list(scr[3 + _NBUF:3 + 2 * _NBUF])
    ssem = list(scr[3 + 2 * _NBUF:3 + 3 * _NBUF])
    acc = scr[3 + 3 * _NBUF]
    c = lax.axis_index("c")
    s = lax.axis_index("s")

    def zero_acc():
        for k in range(_RPT // _CH):
            pltpu.sync_copy(zbuf, acc.at[pl.ds(s * _RPT + k * _CH, _CH)])

    def pipeline(xws):
        for g in range(_LAG):
            pltpu.async_copy(xws.at[srcv.at[g]], rows[g], gsem[g])

        def body(i, carry):
            for b in range(_NBUF):
                j = _NBUF * i + b
                b2 = (b + _LAG) % _NBUF

                @pl.when((j >= _NBUF - _LAG) & (j + _LAG < _CPT))
                def _():
                    pltpu.make_async_copy(
                        rows[b2], acc.at[dstv.at[j]], ssem[b2]).wait()

                @pl.when(j + _LAG < _CPT)
                def _():
                    pltpu.async_copy(
                        xws.at[srcv.at[j + _LAG]], rows[b2], gsem[b2])

                pltpu.make_async_copy(
                    xws.at[srcv.at[j]], rows[b], gsem[b]).wait()
                pltpu.async_copy(
                    rows[b], acc.at[dstv.at[j]], ssem[b], add=True)
            return carry

        lax.fori_loop(0, _CPT // _NBUF, body, 0)
        for b in range(_NBUF):
            pltpu.make_async_copy(
                rows[b], acc.at[dstv.at[0]], ssem[b]).wait()

    def run(xws1, xws2, src3d, dst3d, out1, out2):
        pltpu.sync_copy(src3d.at[s], srcv)
        pltpu.sync_copy(dst3d.at[s], dstv)
        z16 = jnp.zeros((16,), jnp.float32)

        def zbody(r, carry):
            for cc in range(_HD // 16):
                zbuf[r, pl.ds(cc * 16, 16)] = z16
            return carry

        lax.fori_loop(0, _CH, zbody, 0)
        zero_acc()
        plsc.subcore_barrier()
        pipeline(xws1)
        plsc.subcore_barrier()
        pltpu.sync_copy(acc.at[pl.ds(s * _RPT, _RPT)],
                        out1.at[pl.ds(s * _RPT, _RPT)])
        zero_acc()
        plsc.subcore_barrier()
        pipeline(xws2)
        plsc.subcore_barrier()
        pltpu.sync_copy(acc.at[pl.ds(s * _RPT, _RPT)],
                        out2.at[pl.ds(s * _RPT, _RPT)])

    @pl.when(c == 0)
    def _():
        run(xa1, xa2, src_a, dst_a, out_a1, out_a2)

    @pl.when(c == 1)
    def _():
        run(xf1, xf2, src_f, dst_f, out_f1, out_f2)


def _t1_body(xa_ref, wa_ref, dega_ref, xf_ref, wf_ref, degf_ref,
             oa1_ref, oa2_ref, of1_ref, of2_ref):
    sa = lax.rsqrt(jnp.maximum(dega_ref[0, 0, :], 1.0))
    xwa = jnp.dot(xa_ref[...], wa_ref[...], preferred_element_type=jnp.float32)
    xwa = xwa * sa[:, None]
    oa1_ref[...] = xwa[:, :_HD]
    oa2_ref[...] = xwa[:, _HD:]
    sf = lax.rsqrt(jnp.maximum(degf_ref[0, 0, :], 1.0))
    xwf = jnp.dot(xf_ref[...], wf_ref[...], preferred_element_type=jnp.float32)
    xwf = xwf * sf[:, None]
    of1_ref[...] = xwf[:, :_HD]
    of2_ref[...] = xwf[:, _HD:]


def _t1(xa, wa, dega3, xf, wf, degf3):
    return pl.pallas_call(
        _t1_body,
        grid=(_NB,),
        in_specs=[
            pl.BlockSpec((_BR, _D), lambda i: (i, 0)),
            pl.BlockSpec((_D, _D), lambda i: (0, 0)),
            pl.BlockSpec((1, 1, _BR), lambda i: (i, 0, 0)),
            pl.BlockSpec((_BR, _D), lambda i: (i, 0)),
            pl.BlockSpec((_D, _D), lambda i: (0, 0)),
            pl.BlockSpec((1, 1, _BR), lambda i: (i, 0, 0)),
        ],
        out_specs=[pl.BlockSpec((_BR, _HD), lambda i: (i, 0))] * 4,
        out_shape=[_SDS((_N, _HD), jnp.float32)] * 4,
    )(xa, wa, dega3, xf, wf, degf3)


def _t2_body(a1_ref, a2_ref, dega_ref, b1aa_ref, b1ab_ref,
             f1_ref, f2_ref, degf_ref, b1fa_ref, b1fb_ref,
             wl1a_ref, wl1b_ref, bl1_ref,
             enca_ref, encf_ref, esuma_ref, esumf_ref):
    i = pl.program_id(0)
    row = i * _BR2 + lax.broadcasted_iota(jnp.int32, (_BR2, 1), 0)
    valid = row < _N
    sa = lax.rsqrt(jnp.maximum(dega_ref[0, 0, :], 1.0))
    ha1 = jnp.maximum(a1_ref[...] * sa[:, None] + b1aa_ref[...], 0.0)
    ha2 = jnp.maximum(a2_ref[...] * sa[:, None] + b1ab_ref[...], 0.0)
    ea = (jnp.dot(ha1, wl1a_ref[...], preferred_element_type=jnp.float32)
          + jnp.dot(ha2, wl1b_ref[...], preferred_element_type=jnp.float32)
          + bl1_ref[...])
    enca_ref[...] = ea
    sf = lax.rsqrt(jnp.maximum(degf_ref[0, 0, :], 1.0))
    hf1 = jnp.maximum(f1_ref[...] * sf[:, None] + b1fa_ref[...], 0.0)
    hf2 = jnp.maximum(f2_ref[...] * sf[:, None] + b1fb_ref[...], 0.0)
    ef = (jnp.dot(hf1, wl1a_ref[...], preferred_element_type=jnp.float32)
          + jnp.dot(hf2, wl1b_ref[...], preferred_element_type=jnp.float32)
          + bl1_ref[...])
    encf_ref[...] = ef

    @pl.when(i == 0)
    def _():
        esuma_ref[...] = jnp.zeros_like(esuma_ref)
        esumf_ref[...] = jnp.zeros_like(esumf_ref)

    zero = jnp.zeros_like(ea)
    esuma_ref[...] += jnp.sum(jnp.where(valid, ea, zero), axis=0,
                              keepdims=True)
    esumf_ref[...] += jnp.sum(jnp.where(valid, ef, zero), axis=0,
                              keepdims=True)


def _t2(a1, a2, dega3, b1a, f1, f2, degf3, b1f, wl1, bl1):
    half = pl.BlockSpec((_BR2, _HD), lambda i: (i, 0))
    deg = pl.BlockSpec((1, 1, _BR2), lambda i: (i, 0, 0))
    vhd = pl.BlockSpec((1, _HD), lambda i: (0, 0))
    wsp = pl.BlockSpec((_HD, _U), lambda i: (0, 0))
    vu = pl.BlockSpec((1, _U), lambda i: (0, 0))
    return pl.pallas_call(
        _t2_body,
        grid=(_NB2,),
        in_specs=[half, half, deg, vhd, vhd,
                  half, half, deg, vhd, vhd,
                  wsp, wsp, vu],
        out_specs=[
            pl.BlockSpec((_BR2, _U), lambda i: (i, 0)),
            pl.BlockSpec((_BR2, _U), lambda i: (i, 0)),
            vu, vu,
        ],
        out_shape=[_SDS((_NPAD, _U), jnp.float32),
                   _SDS((_NPAD, _U), jnp.float32),
                   _SDS((1, _U), jnp.float32), _SDS((1, _U), jnp.float32)],
    )(a1, a2, dega3, b1a[:, :_HD], b1a[:, _HD:],
      f1, f2, degf3, b1f[:, :_HD], b1f[:, _HD:],
      wl1[:_HD], wl1[_HD:], bl1)


def _t3_body(enca_ref, sumf_ref, encf_ref, suma_ref, wl2_ref, bl2_ref,
             wa2_ref, wf2_ref, dega_ref, degf_ref,
             oa1_ref, oa2_ref, of1_ref, of2_ref):
    ea = enca_ref[...] + 0.1 * sumf_ref[...]
    deca = jnp.dot(ea, wl2_ref[...], preferred_element_type=jnp.float32) \
        + bl2_ref[...]
    xwa = jnp.dot(deca, wa2_ref[...], preferred_element_type=jnp.float32)
    sa = lax.rsqrt(jnp.maximum(dega_ref[0, 0, :], 1.0))
    xwa = xwa * sa[:, None]
    oa1_ref[...] = xwa[:, :_HD]
    oa2_ref[...] = xwa[:, _HD:]
    ef = encf_ref[...] + 0.1 * suma_ref[...]
    decf = jnp.dot(ef, wl2_ref[...], preferred_element_type=jnp.float32) \
        + bl2_ref[...]
    xwf = jnp.dot(decf, wf2_ref[...], preferred_element_type=jnp.float32)
    sf = lax.rsqrt(jnp.maximum(degf_ref[0, 0, :], 1.0))
    xwf = xwf * sf[:, None]
    of1_ref[...] = xwf[:, :_HD]
    of2_ref[...] = xwf[:, _HD:]


def _t3(enca, sumf, encf, suma, wl2, bl2, wa2, wf2, dega3, degf3):
    enc = pl.BlockSpec((_BR2, _U), lambda i: (i, 0))
    vu = pl.BlockSpec((1, _U), lambda i: (0, 0))
    wl2s = pl.BlockSpec((_U, _D), lambda i: (0, 0))
    vd = pl.BlockSpec((1, _D), lambda i: (0, 0))
    wsq = pl.BlockSpec((_D, _D), lambda i: (0, 0))
    deg = pl.BlockSpec((1, 1, _BR2), lambda i: (i, 0, 0))
    outs = pl.BlockSpec((_BR2, _HD), lambda i: (i, 0))
    return pl.pallas_call(
        _t3_body,
        grid=(_NB2,),
        in_specs=[enc, vu, enc, vu, wl2s, vd, wsq, wsq, deg, deg],
        out_specs=[outs] * 4,
        out_shape=[_SDS((_NPAD, _HD), jnp.float32)] * 4,
    )(enca, sumf, encf, suma, wl2, bl2, wa2, wf2, dega3, degf3)


def _t4_body(aa1_ref, aa2_ref, dega_ref, b2aa_ref, b2ab_ref,
             af1_ref, af2_ref, degf_ref, b2fa_ref, b2fb_ref,
             wca_ref, wcb_ref, bc_ref, o_ref,
             asum1_ref, asum2_ref, fmax1_ref, fmax2_ref):
    i = pl.program_id(0)
    row = i * _BR2 + lax.broadcasted_iota(jnp.int32, (_BR2, 1), 0)
    valid = row < _N
    ninf = jnp.float32(-jnp.inf)
    sa = lax.rsqrt(jnp.maximum(dega_ref[0, 0, :], 1.0))
    ha1 = jnp.maximum(aa1_ref[...] * sa[:, None] + b2aa_ref[...], 0.0)
    ha2 = jnp.maximum(aa2_ref[...] * sa[:, None] + b2ab_ref[...], 0.0)
    sf = lax.rsqrt(jnp.maximum(degf_ref[0, 0, :], 1.0))
    hf1 = jnp.maximum(af1_ref[...] * sf[:, None] + b2fa_ref[...], 0.0)
    hf2 = jnp.maximum(af2_ref[...] * sf[:, None] + b2fb_ref[...], 0.0)
    zero = jnp.zeros_like(ha1)
    ha1 = jnp.where(valid, ha1, zero)
    ha2 = jnp.where(valid, ha2, zero)
    hf1 = jnp.where(valid, hf1, ninf)
    hf2 = jnp.where(valid, hf2, ninf)

    @pl.when(i == 0)
    def _():
        asum1_ref[...] = jnp.zeros_like(asum1_ref)
        asum2_ref[...] = jnp.zeros_like(asum2_ref)
        fmax1_ref[...] = jnp.full_like(fmax1_ref, -jnp.inf)
        fmax2_ref[...] = jnp.full_like(fmax2_ref, -jnp.inf)

    asum1_ref[...] += jnp.sum(ha1, axis=0, keepdims=True)
    asum2_ref[...] += jnp.sum(ha2, axis=0, keepdims=True)
    fmax1_ref[...] = jnp.maximum(fmax1_ref[...],
                                 jnp.max(hf1, axis=0, keepdims=True))
    fmax2_ref[...] = jnp.maximum(fmax2_ref[...],
                                 jnp.max(hf2, axis=0, keepdims=True))

    @pl.when(i == _NB2 - 1)
    def _():
        def norm2(v1, v2):
            m = (jnp.sum(v1) + jnp.sum(v2)) / _D
            ss = jnp.sum((v1 - m) ** 2) + jnp.sum((v2 - m) ** 2)
            sd = jnp.sqrt(ss / (_D - 1))
            w1 = (v1 - m) / sd
            w2 = (v2 - m) / sd
            mn = jnp.minimum(jnp.min(w1), jnp.min(w2))
            mx = jnp.maximum(jnp.max(w1), jnp.max(w2))
            return (w1 - mn) / (mx - mn), (w2 - mn) / (mx - mn)

        na1, na2 = norm2(asum1_ref[...] / _N, asum2_ref[...] / _N)
        nf1, nf2 = norm2(fmax1_ref[...], fmax2_ref[...])
        e1 = na1 + nf1
        e2 = na2 + nf2
        o_ref[...] = (jnp.dot(e1, wca_ref[...],
                              preferred_element_type=jnp.float32)
                      + jnp.dot(e2, wcb_ref[...],
                                preferred_element_type=jnp.float32)
                      + bc_ref[...])


def _t4(aa1, aa2, dega3, b2a, af1, af2, degf3, b2f, wc, bc):
    half = pl.BlockSpec((_BR2, _HD), lambda i: (i, 0))
    deg = pl.BlockSpec((1, 1, _BR2), lambda i: (i, 0, 0))
    vhd = pl.BlockSpec((1, _HD), lambda i: (0, 0))
    return pl.pallas_call(
        _t4_body,
        grid=(_NB2,),
        in_specs=[half, half, deg, vhd, vhd,
                  half, half, deg, vhd, vhd,
                  pl.BlockSpec((_HD, _D), lambda i: (0, 0)),
                  pl.BlockSpec((_HD, _D), lambda i: (0, 0)),
                  pl.BlockSpec((1, _D), lambda i: (0, 0))],
        out_specs=pl.BlockSpec((1, _D), lambda i: (0, 0)),
        out_shape=_SDS((1, _D), jnp.float32),
        scratch_shapes=[
            pltpu.VMEM((1, _HD), jnp.float32),
            pltpu.VMEM((1, _HD), jnp.float32),
            pltpu.VMEM((1, _HD), jnp.float32),
            pltpu.VMEM((1, _HD), jnp.float32),
        ],
    )(aa1, aa2, dega3, b2a[:, :_HD], b2a[:, _HD:],
      af1, af2, degf3, b2f[:, :_HD], b2f[:, _HD:],
      wc[:_HD], wc[_HD:], bc)


def kernel(apig, apig_feat, fcg, fcg_feat,
           W_a1, b_a1, W_a2, b_a2, W_f1, b_f1, W_f2, b_f2,
           W_l1, b_l1, W_l2, b_l2, W_attn, b_attn, W_c, b_c):
    f32 = jnp.float32
    src_a = apig[0].reshape(16, _CPT, _CH)
    dst_a = apig[1].reshape(16, _CPT, _CH)
    src_f = fcg[0].reshape(16, _CPT, _CH)
    dst_f = fcg[1].reshape(16, _CPT, _CH)

    d_as, d_ad, d_fs, d_fd = _deg_call(src_a, dst_a, src_f, dst_f)

    def d3(d):
        return d[:_N].reshape(_NB, 1, _BR)

    def d3p(d):
        return d.reshape(_NB2, 1, _BR2)

    xws_a1, xws_a2, xws_f1, xws_f2 = _t1(apig_feat, W_a1, d3(d_as),
                                         fcg_feat, W_f1, d3(d_fs))
    agg_a1, agg_a2, agg_f1, agg_f2 = _agg_call(
        xws_a1, xws_a2, xws_f1, xws_f2, src_a, dst_a, src_f, dst_f)

    enc_a, enc_f, esum_a, esum_f = _t2(
        agg_a1, agg_a2, d3p(d_ad), b_a1.reshape(1, _D),
        agg_f1, agg_f2, d3p(d_fd), b_f1.reshape(1, _D),
        W_l1, b_l1.reshape(1, _U))

    xws2_a1, xws2_a2, xws2_f1, xws2_f2 = _t3(
        enc_a, esum_f, enc_f, esum_a,
        W_l2, b_l2.reshape(1, _D), W_a2, W_f2, d3p(d_as), d3p(d_fs))
    agg2_a1, agg2_a2, agg2_f1, agg2_f2 = _agg_call(
        xws2_a1, xws2_a2, xws2_f1, xws2_f2, src_a, dst_a, src_f, dst_f)

    wc_pad = jnp.zeros((_D, _D), f32).at[:, :10].set(W_c)
    bc_pad = jnp.zeros((1, _D), f32).at[0, :10].set(b_c)
    out = _t4(agg2_a1, agg2_a2, d3p(d_ad), b_a2.reshape(1, _D),
              agg2_f1, agg2_f2, d3p(d_fd), b_f2.reshape(1, _D),
              wc_pad, bc_pad)
    return out[0, :10]
